# baseline (speedup 1.0000x reference)
import jax
import jax.numpy as jnp
from jax import lax
from jax.experimental import pallas as pl
from jax.experimental.pallas import tpu as pltpu
from jax.experimental.pallas import tpu_sc as plsc

H = 16384
F_IN = 16384
B = 64
S = 16
NNZ = 268435

NCORE = 2
NSUBC = 16
NW = NCORE * NSUBC
CH = 128
NCH = -(-NNZ // (NW * CH))
NNZ_PAD = NW * NCH * CH

ROWS_PER_W = H // NW
ROWS_PER_S = H // NSUBC
NBLK = ROWS_PER_S // CH

_mesh = plsc.VectorSubcoreMesh(core_axis_name="c", subcore_axis_name="s")


def _bcast16(i):
    return jnp.full((16,), i, dtype=jnp.int32)


def _step_body(x_t, p_prev, biasv, ih_cols, ih_rows, ih_vals,
               hh_cols, hh_rows, hh_vals,
               part_out, outprev, hstage,
               acc, gb0, gb1, sb0, sb1, cA, rA, vA, bA,
               gsem0, gsem1, ssem0, ssem1):
    c = lax.axis_index("c")
    s = lax.axis_index("s")
    w = c * NSUBC + s
    r_base = s * ROWS_PER_S

    zero = jnp.zeros((16,), jnp.float32)

    def _zb(i, _):
        for j in range(B // 16):
            sb0[i, pl.ds(j * 16, 16)] = zero
        return 0

    lax.fori_loop(0, CH, _zb, 0, unroll=8)
    for i in range(NBLK):
        pltpu.sync_copy(sb0, acc.at[pl.ds(r_base + i * CH, CH)])

    pltpu.sync_copy(ih_cols.at[w], cA)
    pltpu.sync_copy(ih_rows.at[w], rA)
    pltpu.sync_copy(ih_vals.at[w], vA)
    pltpu.async_copy(x_t.at[cA.at[0]], gb0, gsem0)
    pltpu.sync_copy(biasv.at[pl.ds(r_base, ROWS_PER_S)], bA)

    plsc.subcore_barrier()

    one = jnp.full((16,), 1.0, jnp.float32)
    two = jnp.full((16,), 2.0, jnp.float32)
    for blk in range(NBLK):
        r0 = r_base + blk * CH
        pltpu.sync_copy(p_prev.at[0, pl.ds(r0, CH)], sb0)
        pltpu.sync_copy(p_prev.at[1, pl.ds(r0, CH)], gb1)

        def _row(i, _):
            bv = plsc.load_gather(bA, [_bcast16(blk * CH) + _bcast16(i)])
            for j in range(B // 16):
                z = sb0[i, pl.ds(j * 16, 16)] + gb1[i, pl.ds(j * 16, 16)] + bv
                e = jnp.exp(z + z)
                sb1[i, pl.ds(j * 16, 16)] = one - two / (e + one)
            return 0

        lax.fori_loop(0, CH, _row, 0)
        pltpu.sync_copy(sb1, hstage.at[c, pl.ds(r0, CH)])

        @pl.when(c == 0)
        def _():
            pltpu.sync_copy(sb1, outprev.at[pl.ds(r0, CH)])

    plsc.subcore_barrier()

    gbufs = (gb0, gb1)
    gsems = (gsem0, gsem1)
    sbufs = (sb0, sb1)
    ssems = (ssem0, ssem1)

    def _run_matrix(table, first, cols3, rows3, vals3):
        if not first:
            pltpu.sync_copy(cols3.at[w], cA)
            pltpu.sync_copy(rows3.at[w], rA)
            pltpu.sync_copy(vals3.at[w], vA)
            pltpu.async_copy(table.at[cA.at[0]], gb0, gsem0)

        def _scale(gb, sb, k):
            def _grp(g, _):
                base = _bcast16(k * CH + g * 16)
                for l in range(16):
                    i = g * 16 + l
                    bv = plsc.load_gather(vA, [base + _bcast16(l)])
                    a = [gb[i, pl.ds(j * 16, 16)] for j in range(B // 16)]
                    for j in range(B // 16):
                        sb[i, pl.ds(j * 16, 16)] = a[j] * bv
                return 0

            lax.fori_loop(0, CH // 16, _grp, 0)

        def _iter(ko, _):
            for b in range(2):
                k = ko * 2 + b
                gb, gsem = gbufs[b], gsems[b]
                sb, ssem = sbufs[b], ssems[b]
                pltpu.make_async_copy(table.at[cA.at[k]], gb, gsem).wait()
                if b == 0:
                    pltpu.async_copy(table.at[cA.at[k + 1]], gbufs[1], gsems[1])
                else:
                    @pl.when(ko < NCH // 2 - 1)
                    def _():
                        pltpu.async_copy(table.at[cA.at[k + 1]], gbufs[0], gsems[0])
                @pl.when(ko >= 1)
                def _():
                    pltpu.make_async_copy(sb, acc.at[rA.at[k]], ssem).wait()
                _scale(gb, sb, k)
                pltpu.async_copy(sb, acc.at[rA.at[k]], ssem, add=True)
            return 0

        lax.fori_loop(0, NCH // 2, _iter, 0)
        for b in range(2):
            k = NCH - 2 + b
            pltpu.make_async_copy(sbufs[b], acc.at[rA.at[k]], ssems[b]).wait()

    _run_matrix(x_t, True, ih_cols, ih_rows, ih_vals)
    _run_matrix(hstage.at[c], False, hh_cols, hh_rows, hh_vals)

    plsc.subcore_barrier()
    for i in range(NBLK):
        r = r_base + i * CH
        pltpu.sync_copy(acc.at[pl.ds(r, CH)], part_out.at[c, pl.ds(r, CH)])


_params = pltpu.CompilerParams(needs_layout_passes=False,
                               use_tc_tiling_on_sc=False)

_k1 = pl.kernel(
    _step_body,
    out_type=(
        jax.ShapeDtypeStruct((NCORE, H, B), jnp.float32),
        jax.ShapeDtypeStruct((H, B), jnp.float32),
        jax.ShapeDtypeStruct((NCORE, H, B), jnp.float32),
    ),
    mesh=_mesh,
    compiler_params=_params,
    scratch_types=[
        pltpu.VMEM_SHARED((H, B), jnp.float32),
        pltpu.VMEM((CH, B), jnp.float32),
        pltpu.VMEM((CH, B), jnp.float32),
        pltpu.VMEM((CH, B), jnp.float32),
        pltpu.VMEM((CH, B), jnp.float32),
        pltpu.VMEM((NCH, CH), jnp.int32),
        pltpu.VMEM((NCH, CH), jnp.int32),
        pltpu.VMEM((NCH * CH,), jnp.float32),
        pltpu.VMEM((ROWS_PER_S,), jnp.float32),
        pltpu.SemaphoreType.DMA,
        pltpu.SemaphoreType.DMA,
        pltpu.SemaphoreType.DMA,
        pltpu.SemaphoreType.DMA,
    ],
)


def _tanh_tc_body(p0, p1, b_ih, b_hh, h_out):
    h_out[...] = jnp.tanh(p0[...] + p1[...] + b_ih[...] + b_hh[...])


_k2 = pl.pallas_call(
    _tanh_tc_body,
    grid=(NW,),
    in_specs=[
        pl.BlockSpec((ROWS_PER_W, B), lambda i: (i, 0)),
        pl.BlockSpec((ROWS_PER_W, B), lambda i: (i, 0)),
        pl.BlockSpec((ROWS_PER_W, 1), lambda i: (i, 0)),
        pl.BlockSpec((ROWS_PER_W, 1), lambda i: (i, 0)),
    ],
    out_specs=pl.BlockSpec((ROWS_PER_W, B), lambda i: (i, 0)),
    out_shape=jax.ShapeDtypeStruct((H, B), jnp.float32),
)


def _prep(rows, cols, vals):
    pad = NNZ_PAD - NNZ
    rows = jnp.pad(rows, (0, pad)).reshape(NW, NCH, CH)
    cols = jnp.pad(cols, (0, pad)).reshape(NW, NCH, CH)
    vals = jnp.pad(vals, (0, pad)).reshape(NW, NCH * CH)
    return rows, cols, vals


def kernel(x, ih_vals, bias_ih, hh_vals, bias_hh, ih_rows, ih_cols, hh_rows, hh_cols):
    xp = jnp.transpose(x, (1, 2, 0))
    ihr, ihc, ihv = _prep(ih_rows, ih_cols, ih_vals)
    hhr, hhc, hhv = _prep(hh_rows, hh_cols, hh_vals)
    biasv = (bias_ih + bias_hh).reshape(H)

    p = jnp.stack([
        jnp.broadcast_to(-biasv[:, None], (H, B)),
        jnp.zeros((H, B), jnp.float32),
    ])

    outs = []
    for t in range(S):
        p, outprev, _ = _k1(xp[t], p, biasv, ihc, ihr, ihv, hhc, hhr, hhv)
        if t >= 1:
            outs.append(outprev)
    outs.append(_k2(p[0], p[1], bias_ih, bias_hh))

    out = jnp.transpose(jnp.stack(outs), (2, 0, 1))
    h_final = jnp.transpose(outs[-1][None, :, :], (2, 0, 1))
    return (out, h_final)
